# baseline (speedup 1.0000x reference)
import functools

import jax
import jax.numpy as jnp
from jax import lax
from jax.experimental import pallas as pl
from jax.experimental.pallas import tpu as pltpu
from jax.experimental.pallas import tpu_sc as plsc

N = 10000
NFEAT = 128
NHID = 128
NCLASS = 64
E = 320000

NC = 2
NS = 16
NW = NC * NS
CHUNK = 128
NCH = 80
IB = 8
NBUF = 4
CH2 = 64
TOTCH = EPAD_CH2 = 5120
IB2 = 16
CHA = 320
CHB = 0
EPAD = NW * NCH * CHUNK
RPS = 632
NPAD = NS * RPS

_MESH = plsc.VectorSubcoreMesh(core_axis_name="c", subcore_axis_name="s")


@functools.partial(
    pl.kernel,
    out_type=jax.ShapeDtypeStruct((NC, NPAD, NHID), jnp.float32),
    mesh=_MESH,
    scratch_types=[
        pltpu.VMEM((NCH, CHUNK), jnp.int32),
        pltpu.VMEM((CHUNK, NHID), jnp.float32),
        pltpu.VMEM_SHARED((NPAD, NHID), jnp.float32),
        pltpu.SemaphoreType.DMA,
    ],
)
def _sc_degree(dst_hbm, zeros_hbm, ones_hbm, out_hbm, dst_v, ones_v, acc,
               sem):
    cid = lax.axis_index("c")
    sid = lax.axis_index("s")
    wid = cid * NS + sid
    pltpu.sync_copy(zeros_hbm, acc.at[pl.ds(sid * RPS, RPS)])
    pltpu.sync_copy(dst_hbm.at[wid], dst_v)
    pltpu.sync_copy(ones_hbm, ones_v)
    plsc.subcore_barrier()

    @pl.loop(0, NCH, step=8)
    def _(j):
        descs = [pltpu.async_copy(ones_v, acc.at[dst_v.at[j + k]], sem,
                                  add=True) for k in range(8)]
        for d in descs:
            d.wait()

    plsc.subcore_barrier()
    pltpu.sync_copy(acc.at[pl.ds(sid * RPS, RPS)],
                    out_hbm.at[cid, pl.ds(sid * RPS, RPS)])


def _make_sc_scatter(width):
    @functools.partial(
        pl.kernel,
        out_type=jax.ShapeDtypeStruct((NC, NPAD, width), jnp.float32),
        mesh=_MESH,
        scratch_types=[
            pltpu.VMEM((IB2, CH2), jnp.int32),
            pltpu.VMEM((IB2, CH2), jnp.int32),
            pltpu.VMEM_SHARED((NPAD, width), jnp.float32),
        ] + [pltpu.VMEM((CH2, width), jnp.float32)] * NBUF
          + [pltpu.SemaphoreType.DMA] * NBUF,
    )
    def scat(g_hbm, src_hbm, dst_hbm, zeros_hbm, out_hbm,
             src_v, dst_v, acc, *bufs_sems):
        bufs = bufs_sems[:NBUF]
        sems = bufs_sems[NBUF:]
        cid = lax.axis_index("c")
        sid = lax.axis_index("s")
        wid = cid * NS + sid
        pltpu.sync_copy(zeros_hbm, acc.at[pl.ds(sid * RPS, RPS)])
        plsc.subcore_barrier()

        def gather(i, k):
            return pltpu.make_async_copy(g_hbm.at[src_v.at[i]],
                                         bufs[k], sems[k])

        def prefetch(i, k):
            @pl.when(i < IB2)
            def _():
                gather(i, k).start()

        def run(nchunks, base):
            @pl.loop(0, nchunks // IB2)
            def _(b):
                off = base + b * IB2
                pltpu.sync_copy(src_hbm.at[pl.ds(off, IB2)], src_v)
                pltpu.sync_copy(dst_hbm.at[pl.ds(off, IB2)], dst_v)
                for k in range(NBUF):
                    gather(k, k).start()

                @pl.loop(0, IB2, step=NBUF)
                def _(i):
                    for k in range(NBUF):
                        gather(i + k, k).wait()
                        pltpu.sync_copy(bufs[k], acc.at[dst_v.at[i + k]],
                                        add=True)
                        prefetch(i + NBUF + k, k)

        @pl.when(cid == 0)
        def _():
            run(CHA, sid * CHA)

        if CHB:
            @pl.when(cid == 1)
            def _():
                run(CHB, NS * CHA + sid * CHB)

        plsc.subcore_barrier()
        pltpu.sync_copy(acc.at[pl.ds(sid * RPS, RPS)],
                        out_hbm.at[cid, pl.ds(sid * RPS, RPS)])

    return scat


_sc_scatter_hid = _make_sc_scatter(NHID)


def _mm_body(x_ref, w_ref, o_ref):
    o_ref[...] = jnp.dot(x_ref[...], w_ref[...],
                         preferred_element_type=jnp.float32)


def _dis(d0_ref, d1_ref):
    deg = d0_ref[...][:N, 0:1] + d1_ref[...][:N, 0:1] + 1.0
    return 1.0 / jnp.sqrt(deg)


def _scale_body(d0_ref, d1_ref, h_ref, g_ref):
    g_ref[...] = h_ref[...] * _dis(d0_ref, d1_ref)


def _mid_body(a0_ref, a1_ref, d0_ref, d1_ref, h1_ref, b1_ref,
              h1p_ref, g2_ref):
    dis = _dis(d0_ref, d1_ref)
    acc = a0_ref[...][:N, :] + a1_ref[...][:N, :]
    y = dis * acc + (dis * dis) * h1_ref[...] + b1_ref[...][None, :]
    h1p = jnp.maximum(y, 0.0)
    h1p_ref[...] = h1p
    g2_ref[...] = h1p * dis


def _final_body(a0_ref, a1_ref, d0_ref, d1_ref, h1p_ref, w2_ref, b2_ref,
                o_ref):
    dis = _dis(d0_ref, d1_ref)
    acc = a0_ref[...][:N, :] + a1_ref[...][:N, :]
    t = dis * acc + (dis * dis) * h1p_ref[...]
    z = jnp.dot(t, w2_ref[...],
                preferred_element_type=jnp.float32) + b2_ref[...][None, :]
    m = jnp.max(z, axis=1, keepdims=True)
    s = jnp.sum(jnp.exp(z - m), axis=1, keepdims=True)
    o_ref[...] = z - m - jnp.log(s)


def _f32(shape):
    return jax.ShapeDtypeStruct(shape, jnp.float32)


_tc_matmul = pl.pallas_call(_mm_body, out_shape=_f32((N, NHID)))
_tc_scale = pl.pallas_call(_scale_body, out_shape=_f32((N, NHID)))
_tc_mid = pl.pallas_call(_mid_body,
                         out_shape=(_f32((N, NHID)), _f32((N, NHID))))
_tc_final = pl.pallas_call(_final_body, out_shape=_f32((N, NCLASS)))


def kernel(x, edge_index, W1, b1, W2, b2):
    src = edge_index[0]
    dst = edge_index[1]
    pad = EPAD - E
    srcp = jnp.concatenate(
        [src, jnp.zeros((pad,), src.dtype)]).reshape(NW, NCH, CHUNK)
    dstp = jnp.concatenate(
        [dst, jnp.full((pad,), N, dst.dtype)]).reshape(NW, NCH, CHUNK)
    zh = jnp.zeros((RPS, NHID), jnp.float32)
    oh = jnp.ones((CHUNK, NHID), jnp.float32)

    srcp2 = srcp.reshape(TOTCH, CH2)
    dstp2 = dstp.reshape(TOTCH, CH2)

    degp = _sc_degree(dstp, zh, oh)
    h1 = _tc_matmul(x, W1)
    d0, d1 = degp[0], degp[1]
    g1 = _tc_scale(d0, d1, h1)
    acc1 = _sc_scatter_hid(g1, srcp2, dstp2, zh)
    h1p, g2 = _tc_mid(acc1[0], acc1[1], d0, d1, h1, b1)
    acc2 = _sc_scatter_hid(g2, srcp2, dstp2, zh)
    return _tc_final(acc2[0], acc2[1], d0, d1, h1p, W2, b2)


# speedup vs baseline: 1.2825x; 1.2825x over previous
import functools

import jax
import jax.numpy as jnp
from jax import lax
from jax.experimental import pallas as pl
from jax.experimental.pallas import tpu as pltpu
from jax.experimental.pallas import tpu_sc as plsc

N = 10000
NFEAT = 128
NHID = 128
NCLASS = 64
E = 320000

NC = 2
NS = 16
NW = NC * NS
CHUNK = 128
NCH = 80
IB = 8
NBUF = 4
CH2 = 64
TOTCH = EPAD_CH2 = 5120
IB2 = 16
CHA = 272
CHB = 48
EPAD = NW * NCH * CHUNK
RPS = 632
NPAD = NS * RPS

_MESH = plsc.VectorSubcoreMesh(core_axis_name="c", subcore_axis_name="s")


@functools.partial(
    pl.kernel,
    out_type=jax.ShapeDtypeStruct((NC, NPAD, NHID), jnp.float32),
    mesh=_MESH,
    scratch_types=[
        pltpu.VMEM((NCH, CHUNK), jnp.int32),
        pltpu.VMEM((CHUNK, NHID), jnp.float32),
        pltpu.VMEM_SHARED((NPAD, NHID), jnp.float32),
        pltpu.SemaphoreType.DMA,
    ],
)
def _sc_degree(dst_hbm, zeros_hbm, ones_hbm, out_hbm, dst_v, ones_v, acc,
               sem):
    cid = lax.axis_index("c")
    sid = lax.axis_index("s")
    wid = cid * NS + sid
    pltpu.sync_copy(zeros_hbm, acc.at[pl.ds(sid * RPS, RPS)])
    pltpu.sync_copy(dst_hbm.at[wid], dst_v)
    pltpu.sync_copy(ones_hbm, ones_v)
    plsc.subcore_barrier()

    @pl.loop(0, NCH, step=8)
    def _(j):
        descs = [pltpu.async_copy(ones_v, acc.at[dst_v.at[j + k]], sem,
                                  add=True) for k in range(8)]
        for d in descs:
            d.wait()

    plsc.subcore_barrier()
    pltpu.sync_copy(acc.at[pl.ds(sid * RPS, RPS)],
                    out_hbm.at[cid, pl.ds(sid * RPS, RPS)])


def _make_sc_scatter(width):
    @functools.partial(
        pl.kernel,
        out_type=jax.ShapeDtypeStruct((NC, NPAD, width), jnp.float32),
        mesh=_MESH,
        scratch_types=[
            pltpu.VMEM((IB2, CH2), jnp.int32),
            pltpu.VMEM((IB2, CH2), jnp.int32),
            pltpu.VMEM_SHARED((NPAD, width), jnp.float32),
        ] + [pltpu.VMEM((CH2, width), jnp.float32)] * NBUF
          + [pltpu.SemaphoreType.DMA] * NBUF,
    )
    def scat(g_hbm, src_hbm, dst_hbm, zeros_hbm, out_hbm,
             src_v, dst_v, acc, *bufs_sems):
        bufs = bufs_sems[:NBUF]
        sems = bufs_sems[NBUF:]
        cid = lax.axis_index("c")
        sid = lax.axis_index("s")
        wid = cid * NS + sid
        pltpu.sync_copy(zeros_hbm, acc.at[pl.ds(sid * RPS, RPS)])
        plsc.subcore_barrier()

        def gather(i, k):
            return pltpu.make_async_copy(g_hbm.at[src_v.at[i]],
                                         bufs[k], sems[k])

        def prefetch(i, k):
            @pl.when(i < IB2)
            def _():
                gather(i, k).start()

        def run(nchunks, base):
            @pl.loop(0, nchunks // IB2)
            def _(b):
                off = base + b * IB2
                pltpu.sync_copy(src_hbm.at[pl.ds(off, IB2)], src_v)
                pltpu.sync_copy(dst_hbm.at[pl.ds(off, IB2)], dst_v)
                for k in range(NBUF):
                    gather(k, k).start()

                @pl.loop(0, IB2, step=NBUF)
                def _(i):
                    for k in range(NBUF):
                        gather(i + k, k).wait()
                        pltpu.sync_copy(bufs[k], acc.at[dst_v.at[i + k]],
                                        add=True)
                        prefetch(i + NBUF + k, k)

        @pl.when(cid == 0)
        def _():
            run(CHA, sid * CHA)

        if CHB:
            @pl.when(cid == 1)
            def _():
                run(CHB, NS * CHA + sid * CHB)

        plsc.subcore_barrier()
        pltpu.sync_copy(acc.at[pl.ds(sid * RPS, RPS)],
                        out_hbm.at[cid, pl.ds(sid * RPS, RPS)])

    return scat


_sc_scatter_hid = _make_sc_scatter(NHID)


def _mm_body(x_ref, w_ref, o_ref):
    o_ref[...] = jnp.dot(x_ref[...], w_ref[...],
                         preferred_element_type=jnp.float32)


def _dis(d0_ref, d1_ref):
    deg = d0_ref[...][:N, 0:1] + d1_ref[...][:N, 0:1] + 1.0
    return 1.0 / jnp.sqrt(deg)


def _scale_body(d0_ref, d1_ref, h_ref, g_ref):
    g_ref[...] = h_ref[...] * _dis(d0_ref, d1_ref)


def _mid_body(a0_ref, a1_ref, d0_ref, d1_ref, h1_ref, b1_ref,
              h1p_ref, g2_ref):
    dis = _dis(d0_ref, d1_ref)
    acc = a0_ref[...][:N, :] + a1_ref[...][:N, :]
    y = dis * acc + (dis * dis) * h1_ref[...] + b1_ref[...][None, :]
    h1p = jnp.maximum(y, 0.0)
    h1p_ref[...] = h1p
    g2_ref[...] = h1p * dis


def _final_body(a0_ref, a1_ref, d0_ref, d1_ref, h1p_ref, w2_ref, b2_ref,
                o_ref):
    dis = _dis(d0_ref, d1_ref)
    acc = a0_ref[...][:N, :] + a1_ref[...][:N, :]
    t = dis * acc + (dis * dis) * h1p_ref[...]
    z = jnp.dot(t, w2_ref[...],
                preferred_element_type=jnp.float32) + b2_ref[...][None, :]
    m = jnp.max(z, axis=1, keepdims=True)
    s = jnp.sum(jnp.exp(z - m), axis=1, keepdims=True)
    o_ref[...] = z - m - jnp.log(s)


def _f32(shape):
    return jax.ShapeDtypeStruct(shape, jnp.float32)


_tc_matmul = pl.pallas_call(_mm_body, out_shape=_f32((N, NHID)))
_tc_scale = pl.pallas_call(_scale_body, out_shape=_f32((N, NHID)))
_tc_mid = pl.pallas_call(_mid_body,
                         out_shape=(_f32((N, NHID)), _f32((N, NHID))))
_tc_final = pl.pallas_call(_final_body, out_shape=_f32((N, NCLASS)))


def kernel(x, edge_index, W1, b1, W2, b2):
    src = edge_index[0]
    dst = edge_index[1]
    pad = EPAD - E
    srcp = jnp.concatenate(
        [src, jnp.zeros((pad,), src.dtype)]).reshape(NW, NCH, CHUNK)
    dstp = jnp.concatenate(
        [dst, jnp.full((pad,), N, dst.dtype)]).reshape(NW, NCH, CHUNK)
    zh = jnp.zeros((RPS, NHID), jnp.float32)
    oh = jnp.ones((CHUNK, NHID), jnp.float32)

    srcp2 = srcp.reshape(TOTCH, CH2)
    dstp2 = dstp.reshape(TOTCH, CH2)

    degp = _sc_degree(dstp, zh, oh)
    h1 = _tc_matmul(x, W1)
    d0, d1 = degp[0], degp[1]
    g1 = _tc_scale(d0, d1, h1)
    acc1 = _sc_scatter_hid(g1, srcp2, dstp2, zh)
    h1p, g2 = _tc_mid(acc1[0], acc1[1], d0, d1, h1, b1)
    acc2 = _sc_scatter_hid(g2, srcp2, dstp2, zh)
    return _tc_final(acc2[0], acc2[1], d0, d1, h1p, W2, b2)


# speedup vs baseline: 1.4170x; 1.1049x over previous
import functools

import jax
import jax.numpy as jnp
from jax import lax
from jax.experimental import pallas as pl
from jax.experimental.pallas import tpu as pltpu
from jax.experimental.pallas import tpu_sc as plsc

N = 10000
NFEAT = 128
NHID = 128
NCLASS = 64
E = 320000

NC = 2
NS = 16
NW = NC * NS
CHUNK = 128
NCH = 80
IB = 8
NBUF = 4
CH2 = 64
TOTCH = EPAD_CH2 = 5120
IB2 = 16
CHA = 288
CHB = 32
EPAD = NW * NCH * CHUNK
RPS = 632
NPAD = NS * RPS

_MESH = plsc.VectorSubcoreMesh(core_axis_name="c", subcore_axis_name="s")


@functools.partial(
    pl.kernel,
    out_type=jax.ShapeDtypeStruct((NC, NPAD, NHID), jnp.float32),
    mesh=_MESH,
    scratch_types=[
        pltpu.VMEM((NCH, CHUNK), jnp.int32),
        pltpu.VMEM((CHUNK, NHID), jnp.float32),
        pltpu.VMEM_SHARED((NPAD, NHID), jnp.float32),
        pltpu.SemaphoreType.DMA,
    ],
)
def _sc_degree(dst_hbm, zeros_hbm, ones_hbm, out_hbm, dst_v, ones_v, acc,
               sem):
    cid = lax.axis_index("c")
    sid = lax.axis_index("s")
    wid = cid * NS + sid
    pltpu.sync_copy(zeros_hbm, acc.at[pl.ds(sid * RPS, RPS)])
    pltpu.sync_copy(dst_hbm.at[wid], dst_v)
    pltpu.sync_copy(ones_hbm, ones_v)
    plsc.subcore_barrier()

    @pl.loop(0, NCH, step=8)
    def _(j):
        descs = [pltpu.async_copy(ones_v, acc.at[dst_v.at[j + k]], sem,
                                  add=True) for k in range(8)]
        for d in descs:
            d.wait()

    plsc.subcore_barrier()
    pltpu.sync_copy(acc.at[pl.ds(sid * RPS, RPS)],
                    out_hbm.at[cid, pl.ds(sid * RPS, RPS)])


def _make_sc_scatter(width):
    @functools.partial(
        pl.kernel,
        out_type=jax.ShapeDtypeStruct((NC, NPAD, width), jnp.float32),
        mesh=_MESH,
        scratch_types=[
            pltpu.VMEM((IB2, CH2), jnp.int32),
            pltpu.VMEM((IB2, CH2), jnp.int32),
            pltpu.VMEM_SHARED((NPAD, width), jnp.float32),
        ] + [pltpu.VMEM((CH2, width), jnp.float32)] * NBUF
          + [pltpu.SemaphoreType.DMA] * NBUF,
    )
    def scat(g_hbm, src_hbm, dst_hbm, zeros_hbm, out_hbm,
             src_v, dst_v, acc, *bufs_sems):
        bufs = bufs_sems[:NBUF]
        sems = bufs_sems[NBUF:]
        cid = lax.axis_index("c")
        sid = lax.axis_index("s")
        wid = cid * NS + sid
        pltpu.sync_copy(zeros_hbm, acc.at[pl.ds(sid * RPS, RPS)])
        plsc.subcore_barrier()

        def gather(i, k):
            return pltpu.make_async_copy(g_hbm.at[src_v.at[i]],
                                         bufs[k], sems[k])

        def prefetch(i, k):
            @pl.when(i < IB2)
            def _():
                gather(i, k).start()

        def run(nchunks, base):
            @pl.loop(0, nchunks // IB2)
            def _(b):
                off = base + b * IB2
                pltpu.sync_copy(src_hbm.at[pl.ds(off, IB2)], src_v)
                pltpu.sync_copy(dst_hbm.at[pl.ds(off, IB2)], dst_v)
                for k in range(NBUF):
                    gather(k, k).start()

                @pl.loop(0, IB2, step=NBUF)
                def _(i):
                    for k in range(NBUF):
                        gather(i + k, k).wait()
                        pltpu.sync_copy(bufs[k], acc.at[dst_v.at[i + k]],
                                        add=True)
                        prefetch(i + NBUF + k, k)

        @pl.when(cid == 0)
        def _():
            run(CHA, sid * CHA)

        if CHB:
            @pl.when(cid == 1)
            def _():
                run(CHB, NS * CHA + sid * CHB)

        plsc.subcore_barrier()
        pltpu.sync_copy(acc.at[pl.ds(sid * RPS, RPS)],
                        out_hbm.at[cid, pl.ds(sid * RPS, RPS)])

    return scat


_sc_scatter_hid = _make_sc_scatter(NHID)


def _mm_body(x_ref, w_ref, o_ref):
    o_ref[...] = jnp.dot(x_ref[...], w_ref[...],
                         preferred_element_type=jnp.float32)


def _dis(d0_ref, d1_ref):
    deg = d0_ref[...][:N, 0:1] + d1_ref[...][:N, 0:1] + 1.0
    return 1.0 / jnp.sqrt(deg)


def _scale_body(d0_ref, d1_ref, h_ref, g_ref):
    g_ref[...] = h_ref[...] * _dis(d0_ref, d1_ref)


def _mid_body(a0_ref, a1_ref, d0_ref, d1_ref, h1_ref, b1_ref,
              h1p_ref, g2_ref):
    dis = _dis(d0_ref, d1_ref)
    acc = a0_ref[...][:N, :] + a1_ref[...][:N, :]
    y = dis * acc + (dis * dis) * h1_ref[...] + b1_ref[...][None, :]
    h1p = jnp.maximum(y, 0.0)
    h1p_ref[...] = h1p
    g2_ref[...] = h1p * dis


def _final_body(a0_ref, a1_ref, d0_ref, d1_ref, h1p_ref, w2_ref, b2_ref,
                o_ref):
    dis = _dis(d0_ref, d1_ref)
    acc = a0_ref[...][:N, :] + a1_ref[...][:N, :]
    t = dis * acc + (dis * dis) * h1p_ref[...]
    z = jnp.dot(t, w2_ref[...],
                preferred_element_type=jnp.float32) + b2_ref[...][None, :]
    m = jnp.max(z, axis=1, keepdims=True)
    s = jnp.sum(jnp.exp(z - m), axis=1, keepdims=True)
    o_ref[...] = z - m - jnp.log(s)


def _f32(shape):
    return jax.ShapeDtypeStruct(shape, jnp.float32)


_tc_matmul = pl.pallas_call(_mm_body, out_shape=_f32((N, NHID)))
_tc_scale = pl.pallas_call(_scale_body, out_shape=_f32((N, NHID)))
_tc_mid = pl.pallas_call(_mid_body,
                         out_shape=(_f32((N, NHID)), _f32((N, NHID))))
_tc_final = pl.pallas_call(_final_body, out_shape=_f32((N, NCLASS)))


def kernel(x, edge_index, W1, b1, W2, b2):
    src = edge_index[0]
    dst = edge_index[1]
    pad = EPAD - E
    srcp = jnp.concatenate(
        [src, jnp.zeros((pad,), src.dtype)]).reshape(NW, NCH, CHUNK)
    dstp = jnp.concatenate(
        [dst, jnp.full((pad,), N, dst.dtype)]).reshape(NW, NCH, CHUNK)
    zh = jnp.zeros((RPS, NHID), jnp.float32)
    oh = jnp.ones((CHUNK, NHID), jnp.float32)

    srcp2 = srcp.reshape(TOTCH, CH2)
    dstp2 = dstp.reshape(TOTCH, CH2)

    degp = _sc_degree(dstp, zh, oh)
    h1 = _tc_matmul(x, W1)
    d0, d1 = degp[0], degp[1]
    g1 = _tc_scale(d0, d1, h1)
    acc1 = _sc_scatter_hid(g1, srcp2, dstp2, zh)
    h1p, g2 = _tc_mid(acc1[0], acc1[1], d0, d1, h1, b1)
    acc2 = _sc_scatter_hid(g2, srcp2, dstp2, zh)
    return _tc_final(acc2[0], acc2[1], d0, d1, h1p, W2, b2)
